# Initial kernel scaffold; baseline (speedup 1.0000x reference)
import jax
import jax.numpy as jnp
from jax.experimental import pallas as pl


def kernel(v0, generation_mask, t, alpha_bars, X, hist, stddevs, approx_mask):
    raise NotImplementedError("write your pallas kernel here")


# speedup vs baseline: 1.6360x; 1.6360x over previous
import math

import jax
import jax.numpy as jnp
import numpy as np
from jax.experimental import pallas as pl
from jax.experimental.pallas import tpu as pltpu

N_ROWS = 64
L_TOK = 512
NUM_BINS = 8192
CHUNK = 512
N_CHUNKS = NUM_BINS // CHUNK
PI = math.pi
STEP = np.float32(PI / (NUM_BINS - 1))
TINY = np.float32(np.finfo(np.float32).tiny)
SPAN = np.float32(np.float32(1.0) - TINY)
NEG_INF = np.float32(-np.inf)


def _np_threefry2x32(k0, k1, x0, x1):
    rot_a = (13, 15, 26, 6)
    rot_b = (17, 29, 16, 24)
    ks = (np.uint32(k0), np.uint32(k1),
          np.uint32(k0) ^ np.uint32(k1) ^ np.uint32(0x1BD11BDA))
    x0 = (np.uint32(x0) + ks[0]).astype(np.uint32)
    x1 = (np.uint32(x1) + ks[1]).astype(np.uint32)

    def rounds(x0, x1, rots):
        for r in rots:
            x0 = (x0 + x1).astype(np.uint32)
            x1 = ((x1 << np.uint32(r)) | (x1 >> np.uint32(32 - r))).astype(np.uint32)
            x1 = x0 ^ x1
        return x0, x1

    sched = ((rot_a, 1, 2, 1), (rot_b, 2, 0, 2), (rot_a, 0, 1, 3),
             (rot_b, 1, 2, 4), (rot_a, 2, 0, 5))
    for rots, ia, ib, inc in sched:
        x0, x1 = rounds(x0, x1, rots)
        x0 = (x0 + ks[ia]).astype(np.uint32)
        x1 = (x1 + ks[ib] + np.uint32(inc)).astype(np.uint32)
    return x0, x1


def _np_split(kd, num):
    f = np.arange(num, dtype=np.uint64)
    c1 = (f >> np.uint64(32)).astype(np.uint32)
    c2 = (f & np.uint64(0xFFFFFFFF)).astype(np.uint32)
    b1, b2 = _np_threefry2x32(kd[0], kd[1], c1, c2)
    return [(int(b1[i]), int(b2[i])) for i in range(num)]


_KEY = (0, 42)
_K1, _K2 = _np_split(_KEY, 2)
_KCAT, _KUNI, _KGAU = _np_split(_K2, 3)
KS0 = np.uint32(_KCAT[0])
KS1 = np.uint32(_KCAT[1])
KS2 = np.uint32(KS0 ^ KS1 ^ np.uint32(0x1BD11BDA))
_INJ = (
    (KS1, np.uint32((int(KS2) + 1) % (1 << 32))),
    (KS2, np.uint32((int(KS0) + 2) % (1 << 32))),
    (KS0, np.uint32((int(KS1) + 3) % (1 << 32))),
    (KS1, np.uint32((int(KS2) + 4) % (1 << 32))),
    (KS2, np.uint32((int(KS0) + 5) % (1 << 32))),
)
_ROTS = ((13, 15, 26, 6), (17, 29, 16, 24), (13, 15, 26, 6),
         (17, 29, 16, 24), (13, 15, 26, 6))


def _rotl(x, r):
    return jax.lax.shift_left(x, np.uint32(r)) | jax.lax.shift_right_logical(
        x, np.uint32(32 - r))


def _gumbel_from_counts(cnt_u32):
    x0 = jnp.full_like(cnt_u32, KS0)
    x1 = cnt_u32 + KS1
    for rots, (inj0, inj1) in zip(_ROTS, _INJ):
        for r in rots:
            x0 = x0 + x1
            x1 = _rotl(x1, r)
            x1 = x0 ^ x1
        x0 = x0 + inj0
        x1 = x1 + inj1
    bits = x0 ^ x1
    fb = jax.lax.shift_right_logical(bits, np.uint32(9)) | np.uint32(0x3F800000)
    fl = jax.lax.bitcast_convert_type(fb, jnp.float32) - jnp.float32(1.0)
    uu = jnp.maximum(TINY, fl * SPAN + TINY)
    return -jnp.log(-jnp.log(uu))


def _argmax_kernel(t_ref, approx_ref, live_ref, logp_ref, out_ref):
    n = pl.program_id(0)
    row_base = n * (L_TOK * (NUM_BINS - 1))

    @pl.when(approx_ref[n] == 0)
    def _():
        l_iota = jax.lax.broadcasted_iota(jnp.int32, (L_TOK, CHUNK), 0)
        j_iota = jax.lax.broadcasted_iota(jnp.int32, (L_TOK, CHUNK), 1)
        tok_base = row_base + jax.lax.shift_left(l_iota, 13) - l_iota

        def body(c, carry):
            best_v, best_i = carry
            j0 = c * CHUNK
            cnt = (tok_base + (j0 + j_iota)).astype(jnp.uint32)
            g = _gumbel_from_counts(cnt)
            logp = logp_ref[0, 0, pl.ds(j0, CHUNK)]
            v = g + logp[None, :]
            cmax = jnp.max(v, axis=1, keepdims=True)
            jglob = j0 + j_iota
            cidx = jnp.min(jnp.where(v == cmax, jglob, np.int32(1 << 30)),
                           axis=1, keepdims=True)
            upd = cmax > best_v
            return (jnp.where(upd, cmax, best_v),
                    jnp.where(upd, cidx, best_i))

        best_v0 = jnp.full((L_TOK, 1), NEG_INF, jnp.float32)
        best_i0 = jnp.zeros((L_TOK, 1), jnp.int32)
        _, best_i = jax.lax.fori_loop(0, N_CHUNKS, body, (best_v0, best_i0))
        out_ref[0] = best_i


def _bf16(x):
    return x.astype(jnp.bfloat16).astype(jnp.float32)


def _rot_from_vec(wx, wy, wz):
    th = jnp.sqrt(wx * wx + wy * wy + wz * wz)
    x = wx / (th + 1e-12)
    y = wy / (th + 1e-12)
    z = wz / (th + 1e-12)
    bx = _bf16(x)
    by = _bf16(y)
    bz = _bf16(z)
    s = jnp.sin(th)
    c1 = 1.0 - jnp.cos(th)
    b00 = (-(bz * bz)) + (-(by * by))
    b11 = (-(bz * bz)) + (-(bx * bx))
    b22 = (-(by * by)) + (-(bx * bx))
    bxy = bx * by
    bxz = bx * bz
    byz = by * bz
    r00 = 1.0 + c1 * b00
    r01 = (-(s * z)) + c1 * bxy
    r02 = (s * y) + c1 * bxz
    r10 = (s * z) + c1 * bxy
    r11 = 1.0 + c1 * b11
    r12 = (-(s * x)) + c1 * byz
    r20 = (-(s * y)) + c1 * bxz
    r21 = (s * x) + c1 * byz
    r22 = 1.0 + c1 * b22
    return (r00, r01, r02, r10, r11, r12, r20, r21, r22)


def _tail_kernel(bin_ref, ux_ref, uy_ref, uz_ref, unif_ref, gau_ref,
                 v0x_ref, v0y_ref, v0z_ref, std_ref, c0_ref, approx_ref,
                 mask_ref,
                 vnx_ref, vny_ref, vnz_ref, ex_ref, ey_ref, ez_ref):
    bin_idx = bin_ref[:, 0, :]
    idx_f = bin_idx.astype(jnp.float32)
    bs = idx_f * STEP
    bw = (idx_f + 1.0) * STEP - bs
    theta_hist = bs + unif_ref[...] * bw

    std = std_ref[...]
    sg = jnp.abs(2.0 * std + gau_ref[...] * std)
    theta_gauss = sg - jnp.floor(sg * np.float32(1.0 / PI)) * np.float32(PI)
    theta = jnp.where(approx_ref[...] != 0, theta_gauss, theta_hist)

    ux = ux_ref[...]
    uy = uy_ref[...]
    uz = uz_ref[...]
    un = jnp.sqrt(ux * ux + uy * uy + uz * uz) + 1e-12
    scale = theta / un
    ex = ux * scale
    ey = uy * scale
    ez = uz * scale
    ex_ref[...] = ex
    ey_ref[...] = ey
    ez_ref[...] = ez

    e00, e01, e02, e10, e11, e12, e20, e21, e22 = _rot_from_vec(ex, ey, ez)

    c0 = c0_ref[...]
    w0x = c0 * v0x_ref[...]
    w0y = c0 * v0y_ref[...]
    w0z = c0 * v0z_ref[...]
    a00, a01, a02, a10, a11, a12, a20, a21, a22 = _rot_from_vec(w0x, w0y, w0z)

    a00, a01, a02 = _bf16(a00), _bf16(a01), _bf16(a02)
    a10, a11, a12 = _bf16(a10), _bf16(a11), _bf16(a12)
    a20, a21, a22 = _bf16(a20), _bf16(a21), _bf16(a22)
    e00b, e01b, e02b = _bf16(e00), _bf16(e01), _bf16(e02)
    e10b, e11b, e12b = _bf16(e10), _bf16(e11), _bf16(e12)
    e20b, e21b, e22b = _bf16(e20), _bf16(e21), _bf16(e22)

    m00 = (a00 * e00b + a01 * e10b) + a02 * e20b
    m01 = (a00 * e01b + a01 * e11b) + a02 * e21b
    m02 = (a00 * e02b + a01 * e12b) + a02 * e22b
    m10 = (a10 * e00b + a11 * e10b) + a12 * e20b
    m11 = (a10 * e01b + a11 * e11b) + a12 * e21b
    m12 = (a10 * e02b + a11 * e12b) + a12 * e22b
    m20 = (a20 * e00b + a21 * e10b) + a22 * e20b
    m21 = (a20 * e01b + a21 * e11b) + a22 * e21b
    m22 = (a20 * e02b + a21 * e12b) + a22 * e22b

    tr = m00 + m11 + m22
    cos_t = jnp.clip((tr - 1.0) * 0.5, np.float32(-1.0 + 1e-7),
                     np.float32(1.0 - 1e-7))
    th_n = 2.0 * jnp.arctan2(jnp.sqrt(1.0 - cos_t * cos_t), 1.0 + cos_t)
    fac = th_n / (2.0 * jnp.sin(th_n) + 1e-12)
    vnx = (m21 - m12) * fac
    vny = (m02 - m20) * fac
    vnz = (m10 - m01) * fac

    keep = mask_ref[...] != 0
    vnx_ref[...] = jnp.where(keep, vnx, v0x_ref[...])
    vny_ref[...] = jnp.where(keep, vny, v0y_ref[...])
    vnz_ref[...] = jnp.where(keep, vnz, v0z_ref[...])


def kernel(v0, generation_mask, t, alpha_bars, X, hist, stddevs, approx_mask):
    N, L = generation_mask.shape
    t = t.astype(jnp.int32)

    key = jax.random.key(42)
    k1, k2 = jax.random.split(key)
    _, k22, k23 = jax.random.split(k2, 3)
    u = jax.random.normal(k1, (N, L, 3), dtype=jnp.float32)
    unif = jax.random.uniform(k22, (N * L,), jnp.float32).reshape(N, L)
    gau = jax.random.normal(k23, (N * L,), jnp.float32).reshape(N, L)

    logp = jnp.log(hist + 1e-30)
    logp = logp.at[:, NUM_BINS - 1].set(NEG_INF)

    logp_real = logp[:, : NUM_BINS - 1]
    row_max = jnp.max(logp_real, axis=1, keepdims=True)
    chunk_max = jnp.max(logp.reshape(logp.shape[0], N_CHUNKS, CHUNK), axis=2)
    live_tab = (chunk_max >= row_max - 21.0).astype(jnp.int32)
    live_rows = live_tab[t].reshape(N * N_CHUNKS)

    approx_row = approx_mask[t].astype(jnp.int32)

    grid_spec = pltpu.PrefetchScalarGridSpec(
        num_scalar_prefetch=3,
        grid=(N,),
        in_specs=[
            pl.BlockSpec((1, 1, NUM_BINS),
                         lambda n, t_ref, a_ref, l_ref: (t_ref[n], 0, 0)),
        ],
        out_specs=pl.BlockSpec((1, L_TOK, 1), lambda n, t_ref, a_ref, l_ref: (n, 0, 0)),
    )
    bin_idx = pl.pallas_call(
        _argmax_kernel,
        grid_spec=grid_spec,
        out_shape=jax.ShapeDtypeStruct((N, L, 1), jnp.int32),
    )(t, approx_row, live_rows, logp.reshape(logp.shape[0], 1, NUM_BINS))
    bin_idx = bin_idx.reshape(N, 1, L)

    std_b = jnp.broadcast_to(stddevs[t][:, None], (N, L))
    c0_b = jnp.broadcast_to(jnp.sqrt(alpha_bars[t])[:, None], (N, L))
    approx_b = jnp.broadcast_to(approx_row[:, None], (N, L))
    mask_b = generation_mask.astype(jnp.int32)

    outs = pl.pallas_call(
        _tail_kernel,
        out_shape=[jax.ShapeDtypeStruct((N, L), jnp.float32)] * 6,
    )(bin_idx, u[..., 0], u[..., 1], u[..., 2], unif, gau,
      v0[..., 0], v0[..., 1], v0[..., 2], std_b, c0_b, approx_b, mask_b)

    vnx, vny, vnz, ex, ey, ez = outs
    v_noisy = jnp.stack([vnx, vny, vnz], axis=-1)
    e_scaled = jnp.stack([ex, ey, ez], axis=-1)
    return (v_noisy, e_scaled)


# speedup vs baseline: 1.6998x; 1.0390x over previous
import math

import jax
import jax.numpy as jnp
import numpy as np
from jax.experimental import pallas as pl
from jax.experimental.pallas import tpu as pltpu

N_ROWS = 64
L_TOK = 512
NUM_BINS = 8192
CHUNK = 512
N_CHUNKS = NUM_BINS // CHUNK
PI = math.pi
STEP = np.float32(PI / (NUM_BINS - 1))
TINY = np.float32(np.finfo(np.float32).tiny)
SPAN = np.float32(np.float32(1.0) - TINY)
NEG_INF = np.float32(-np.inf)


def _np_threefry2x32(k0, k1, x0, x1):
    rot_a = (13, 15, 26, 6)
    rot_b = (17, 29, 16, 24)
    ks = (np.uint32(k0), np.uint32(k1),
          np.uint32(k0) ^ np.uint32(k1) ^ np.uint32(0x1BD11BDA))
    x0 = (np.uint32(x0) + ks[0]).astype(np.uint32)
    x1 = (np.uint32(x1) + ks[1]).astype(np.uint32)

    def rounds(x0, x1, rots):
        for r in rots:
            x0 = (x0 + x1).astype(np.uint32)
            x1 = ((x1 << np.uint32(r)) | (x1 >> np.uint32(32 - r))).astype(np.uint32)
            x1 = x0 ^ x1
        return x0, x1

    sched = ((rot_a, 1, 2, 1), (rot_b, 2, 0, 2), (rot_a, 0, 1, 3),
             (rot_b, 1, 2, 4), (rot_a, 2, 0, 5))
    for rots, ia, ib, inc in sched:
        x0, x1 = rounds(x0, x1, rots)
        x0 = (x0 + ks[ia]).astype(np.uint32)
        x1 = (x1 + ks[ib] + np.uint32(inc)).astype(np.uint32)
    return x0, x1


def _np_split(kd, num):
    f = np.arange(num, dtype=np.uint64)
    c1 = (f >> np.uint64(32)).astype(np.uint32)
    c2 = (f & np.uint64(0xFFFFFFFF)).astype(np.uint32)
    b1, b2 = _np_threefry2x32(kd[0], kd[1], c1, c2)
    return [(int(b1[i]), int(b2[i])) for i in range(num)]


_KEY = (0, 42)
_K1, _K2 = _np_split(_KEY, 2)
_KCAT, _KUNI, _KGAU = _np_split(_K2, 3)
KS0 = np.uint32(_KCAT[0])
KS1 = np.uint32(_KCAT[1])
KS2 = np.uint32(KS0 ^ KS1 ^ np.uint32(0x1BD11BDA))
_INJ = (
    (KS1, np.uint32((int(KS2) + 1) % (1 << 32))),
    (KS2, np.uint32((int(KS0) + 2) % (1 << 32))),
    (KS0, np.uint32((int(KS1) + 3) % (1 << 32))),
    (KS1, np.uint32((int(KS2) + 4) % (1 << 32))),
    (KS2, np.uint32((int(KS0) + 5) % (1 << 32))),
)
_ROTS = ((13, 15, 26, 6), (17, 29, 16, 24), (13, 15, 26, 6),
         (17, 29, 16, 24), (13, 15, 26, 6))


def _rotl(x, r):
    return jax.lax.shift_left(x, np.uint32(r)) | jax.lax.shift_right_logical(
        x, np.uint32(32 - r))


def _gumbel_from_counts(cnt_u32):
    x0 = jnp.full_like(cnt_u32, KS0)
    x1 = cnt_u32 + KS1
    for rots, (inj0, inj1) in zip(_ROTS, _INJ):
        for r in rots:
            x0 = x0 + x1
            x1 = _rotl(x1, r)
            x1 = x0 ^ x1
        x0 = x0 + inj0
        x1 = x1 + inj1
    bits = x0 ^ x1
    fb = jax.lax.shift_right_logical(bits, np.uint32(9)) | np.uint32(0x3F800000)
    fl = jax.lax.bitcast_convert_type(fb, jnp.float32) - jnp.float32(1.0)
    uu = jnp.maximum(TINY, fl * SPAN + TINY)
    return -jnp.log(-jnp.log(uu))


G_BOUND = np.float32(16.0)


def _argmax_kernel(t_ref, approx_ref, perm_ref, cmax_ref, logp_ref, out_ref):
    n = pl.program_id(0)
    row_base = n * (L_TOK * (NUM_BINS - 1))

    @pl.when(approx_ref[n] == 0)
    def _():
        l_iota = jax.lax.broadcasted_iota(jnp.int32, (L_TOK, CHUNK), 0)
        j_iota = jax.lax.broadcasted_iota(jnp.int32, (L_TOK, CHUNK), 1)
        tok_base = row_base + jax.lax.shift_left(l_iota, 13) - l_iota

        def cond(state):
            i, _, _, rowmin = state
            ii = jnp.minimum(i, N_CHUNKS - 1)
            return jnp.logical_and(i < N_CHUNKS,
                                   cmax_ref[n * N_CHUNKS + ii] + G_BOUND > rowmin)

        def body(state):
            i, best_v, best_i, _ = state
            j0 = perm_ref[n * N_CHUNKS + i] * CHUNK
            cnt = (tok_base + (j0 + j_iota)).astype(jnp.uint32)
            g = _gumbel_from_counts(cnt)
            logp = logp_ref[0, 0, pl.ds(j0, CHUNK)]
            v = g + logp[None, :]
            cmax = jnp.max(v, axis=1, keepdims=True)
            jglob = j0 + j_iota
            cidx = jnp.min(jnp.where(v == cmax, jglob, np.int32(1 << 30)),
                           axis=1, keepdims=True)
            upd = jnp.logical_or(
                cmax > best_v,
                jnp.logical_and(cmax == best_v, cidx < best_i))
            best_v = jnp.where(upd, cmax, best_v)
            best_i = jnp.where(upd, cidx, best_i)
            return (i + 1, best_v, best_i, jnp.min(best_v))

        best_v0 = jnp.full((L_TOK, 1), NEG_INF, jnp.float32)
        best_i0 = jnp.full((L_TOK, 1), np.int32(1 << 30), jnp.int32)
        state = (jnp.int32(0), best_v0, best_i0, NEG_INF)
        _, _, best_i, _ = jax.lax.while_loop(cond, body, state)
        out_ref[0] = best_i


def _bf16(x):
    return x.astype(jnp.bfloat16).astype(jnp.float32)


def _rot_from_vec(wx, wy, wz):
    th = jnp.sqrt(wx * wx + wy * wy + wz * wz)
    x = wx / (th + 1e-12)
    y = wy / (th + 1e-12)
    z = wz / (th + 1e-12)
    bx = _bf16(x)
    by = _bf16(y)
    bz = _bf16(z)
    s = jnp.sin(th)
    c1 = 1.0 - jnp.cos(th)
    b00 = (-(bz * bz)) + (-(by * by))
    b11 = (-(bz * bz)) + (-(bx * bx))
    b22 = (-(by * by)) + (-(bx * bx))
    bxy = bx * by
    bxz = bx * bz
    byz = by * bz
    r00 = 1.0 + c1 * b00
    r01 = (-(s * z)) + c1 * bxy
    r02 = (s * y) + c1 * bxz
    r10 = (s * z) + c1 * bxy
    r11 = 1.0 + c1 * b11
    r12 = (-(s * x)) + c1 * byz
    r20 = (-(s * y)) + c1 * bxz
    r21 = (s * x) + c1 * byz
    r22 = 1.0 + c1 * b22
    return (r00, r01, r02, r10, r11, r12, r20, r21, r22)


def _tail_kernel(bin_ref, ux_ref, uy_ref, uz_ref, unif_ref, gau_ref,
                 v0x_ref, v0y_ref, v0z_ref, std_ref, c0_ref, approx_ref,
                 mask_ref,
                 vnx_ref, vny_ref, vnz_ref, ex_ref, ey_ref, ez_ref):
    bin_idx = bin_ref[:, 0, :]
    idx_f = bin_idx.astype(jnp.float32)
    bs = idx_f * STEP
    bw = (idx_f + 1.0) * STEP - bs
    theta_hist = bs + unif_ref[...] * bw

    std = std_ref[...]
    sg = jnp.abs(2.0 * std + gau_ref[...] * std)
    theta_gauss = sg - jnp.floor(sg * np.float32(1.0 / PI)) * np.float32(PI)
    theta = jnp.where(approx_ref[...] != 0, theta_gauss, theta_hist)

    ux = ux_ref[...]
    uy = uy_ref[...]
    uz = uz_ref[...]
    un = jnp.sqrt(ux * ux + uy * uy + uz * uz) + 1e-12
    scale = theta / un
    ex = ux * scale
    ey = uy * scale
    ez = uz * scale
    ex_ref[...] = ex
    ey_ref[...] = ey
    ez_ref[...] = ez

    e00, e01, e02, e10, e11, e12, e20, e21, e22 = _rot_from_vec(ex, ey, ez)

    c0 = c0_ref[...]
    w0x = c0 * v0x_ref[...]
    w0y = c0 * v0y_ref[...]
    w0z = c0 * v0z_ref[...]
    a00, a01, a02, a10, a11, a12, a20, a21, a22 = _rot_from_vec(w0x, w0y, w0z)

    a00, a01, a02 = _bf16(a00), _bf16(a01), _bf16(a02)
    a10, a11, a12 = _bf16(a10), _bf16(a11), _bf16(a12)
    a20, a21, a22 = _bf16(a20), _bf16(a21), _bf16(a22)
    e00b, e01b, e02b = _bf16(e00), _bf16(e01), _bf16(e02)
    e10b, e11b, e12b = _bf16(e10), _bf16(e11), _bf16(e12)
    e20b, e21b, e22b = _bf16(e20), _bf16(e21), _bf16(e22)

    m00 = (a00 * e00b + a01 * e10b) + a02 * e20b
    m01 = (a00 * e01b + a01 * e11b) + a02 * e21b
    m02 = (a00 * e02b + a01 * e12b) + a02 * e22b
    m10 = (a10 * e00b + a11 * e10b) + a12 * e20b
    m11 = (a10 * e01b + a11 * e11b) + a12 * e21b
    m12 = (a10 * e02b + a11 * e12b) + a12 * e22b
    m20 = (a20 * e00b + a21 * e10b) + a22 * e20b
    m21 = (a20 * e01b + a21 * e11b) + a22 * e21b
    m22 = (a20 * e02b + a21 * e12b) + a22 * e22b

    tr = m00 + m11 + m22
    cos_t = jnp.clip((tr - 1.0) * 0.5, np.float32(-1.0 + 1e-7),
                     np.float32(1.0 - 1e-7))
    th_n = 2.0 * jnp.arctan2(jnp.sqrt(1.0 - cos_t * cos_t), 1.0 + cos_t)
    fac = th_n / (2.0 * jnp.sin(th_n) + 1e-12)
    vnx = (m21 - m12) * fac
    vny = (m02 - m20) * fac
    vnz = (m10 - m01) * fac

    keep = mask_ref[...] != 0
    vnx_ref[...] = jnp.where(keep, vnx, v0x_ref[...])
    vny_ref[...] = jnp.where(keep, vny, v0y_ref[...])
    vnz_ref[...] = jnp.where(keep, vnz, v0z_ref[...])


def kernel(v0, generation_mask, t, alpha_bars, X, hist, stddevs, approx_mask):
    N, L = generation_mask.shape
    t = t.astype(jnp.int32)

    key = jax.random.key(42)
    k1, k2 = jax.random.split(key)
    _, k22, k23 = jax.random.split(k2, 3)
    u = jax.random.normal(k1, (N, L, 3), dtype=jnp.float32)
    unif = jax.random.uniform(k22, (N * L,), jnp.float32).reshape(N, L)
    gau = jax.random.normal(k23, (N * L,), jnp.float32).reshape(N, L)

    logp = jnp.log(hist + 1e-30)
    logp = logp.at[:, NUM_BINS - 1].set(NEG_INF)

    chunk_max = jnp.max(logp.reshape(logp.shape[0], N_CHUNKS, CHUNK), axis=2)
    order = jnp.argsort(-chunk_max, axis=1).astype(jnp.int32)
    cmax_sorted = jnp.take_along_axis(chunk_max, order, axis=1)
    perm_rows = order[t].reshape(N * N_CHUNKS)
    cmax_rows = cmax_sorted[t].reshape(N * N_CHUNKS)

    approx_row = approx_mask[t].astype(jnp.int32)

    grid_spec = pltpu.PrefetchScalarGridSpec(
        num_scalar_prefetch=4,
        grid=(N,),
        in_specs=[
            pl.BlockSpec((1, 1, NUM_BINS),
                         lambda n, t_ref, a_ref, p_ref, c_ref: (t_ref[n], 0, 0)),
        ],
        out_specs=pl.BlockSpec((1, L_TOK, 1),
                               lambda n, t_ref, a_ref, p_ref, c_ref: (n, 0, 0)),
    )
    bin_idx = pl.pallas_call(
        _argmax_kernel,
        grid_spec=grid_spec,
        out_shape=jax.ShapeDtypeStruct((N, L, 1), jnp.int32),
    )(t, approx_row, perm_rows, cmax_rows,
      logp.reshape(logp.shape[0], 1, NUM_BINS))
    bin_idx = bin_idx.reshape(N, 1, L)

    std_b = jnp.broadcast_to(stddevs[t][:, None], (N, L))
    c0_b = jnp.broadcast_to(jnp.sqrt(alpha_bars[t])[:, None], (N, L))
    approx_b = jnp.broadcast_to(approx_row[:, None], (N, L))
    mask_b = generation_mask.astype(jnp.int32)

    outs = pl.pallas_call(
        _tail_kernel,
        out_shape=[jax.ShapeDtypeStruct((N, L), jnp.float32)] * 6,
    )(bin_idx, u[..., 0], u[..., 1], u[..., 2], unif, gau,
      v0[..., 0], v0[..., 1], v0[..., 2], std_b, c0_b, approx_b, mask_b)

    vnx, vny, vnz, ex, ey, ez = outs
    v_noisy = jnp.stack([vnx, vny, vnz], axis=-1)
    e_scaled = jnp.stack([ex, ey, ez], axis=-1)
    return (v_noisy, e_scaled)


# speedup vs baseline: 1.7357x; 1.0212x over previous
import math

import jax
import jax.numpy as jnp
import numpy as np
from jax.experimental import pallas as pl
from jax.experimental.pallas import tpu as pltpu

N_ROWS = 64
L_TOK = 512
NUM_BINS = 8192
CHUNK = 512
N_CHUNKS = NUM_BINS // CHUNK
PI = math.pi
STEP = np.float32(PI / (NUM_BINS - 1))
TINY = np.float32(np.finfo(np.float32).tiny)
SPAN = np.float32(np.float32(1.0) - TINY)
NEG_INF = np.float32(-np.inf)


def _np_threefry2x32(k0, k1, x0, x1):
    rot_a = (13, 15, 26, 6)
    rot_b = (17, 29, 16, 24)
    ks = (np.uint32(k0), np.uint32(k1),
          np.uint32(k0) ^ np.uint32(k1) ^ np.uint32(0x1BD11BDA))
    x0 = (np.uint32(x0) + ks[0]).astype(np.uint32)
    x1 = (np.uint32(x1) + ks[1]).astype(np.uint32)

    def rounds(x0, x1, rots):
        for r in rots:
            x0 = (x0 + x1).astype(np.uint32)
            x1 = ((x1 << np.uint32(r)) | (x1 >> np.uint32(32 - r))).astype(np.uint32)
            x1 = x0 ^ x1
        return x0, x1

    sched = ((rot_a, 1, 2, 1), (rot_b, 2, 0, 2), (rot_a, 0, 1, 3),
             (rot_b, 1, 2, 4), (rot_a, 2, 0, 5))
    for rots, ia, ib, inc in sched:
        x0, x1 = rounds(x0, x1, rots)
        x0 = (x0 + ks[ia]).astype(np.uint32)
        x1 = (x1 + ks[ib] + np.uint32(inc)).astype(np.uint32)
    return x0, x1


def _np_split(kd, num):
    f = np.arange(num, dtype=np.uint64)
    c1 = (f >> np.uint64(32)).astype(np.uint32)
    c2 = (f & np.uint64(0xFFFFFFFF)).astype(np.uint32)
    b1, b2 = _np_threefry2x32(kd[0], kd[1], c1, c2)
    return [(int(b1[i]), int(b2[i])) for i in range(num)]


_KEY = (0, 42)
_K1, _K2 = _np_split(_KEY, 2)
_KCAT, _KUNI, _KGAU = _np_split(_K2, 3)
KS0 = np.uint32(_KCAT[0])
KS1 = np.uint32(_KCAT[1])
KS2 = np.uint32(KS0 ^ KS1 ^ np.uint32(0x1BD11BDA))
_INJ = (
    (KS1, np.uint32((int(KS2) + 1) % (1 << 32))),
    (KS2, np.uint32((int(KS0) + 2) % (1 << 32))),
    (KS0, np.uint32((int(KS1) + 3) % (1 << 32))),
    (KS1, np.uint32((int(KS2) + 4) % (1 << 32))),
    (KS2, np.uint32((int(KS0) + 5) % (1 << 32))),
)
_ROTS = ((13, 15, 26, 6), (17, 29, 16, 24), (13, 15, 26, 6),
         (17, 29, 16, 24), (13, 15, 26, 6))
KS1_I32 = int(np.array(_KCAT[1], dtype=np.uint32).view(np.int32))


def _rotl(x, r):
    return jax.lax.shift_left(x, np.uint32(r)) | jax.lax.shift_right_logical(
        x, np.uint32(32 - r))


def _neg_gumbel_from_x1(x1):
    x0 = x1 + KS0
    rots0 = _ROTS[0]
    x1r = _rotl(x1, rots0[0])
    x1r = x0 ^ x1r
    for r in rots0[1:]:
        x0 = x0 + x1r
        x1r = _rotl(x1r, r)
        x1r = x0 ^ x1r
    x1 = x1r
    x0 = x0 + _INJ[0][0]
    x1 = x1 + _INJ[0][1]
    for rots, (inj0, inj1) in zip(_ROTS[1:], _INJ[1:]):
        for r in rots:
            x0 = x0 + x1
            x1 = _rotl(x1, r)
            x1 = x0 ^ x1
        x0 = x0 + inj0
        x1 = x1 + inj1
    bits = x0 ^ x1
    fb = jax.lax.shift_right_logical(bits, np.uint32(9)) | np.uint32(0x3F800000)
    fl = jax.lax.bitcast_convert_type(fb, jnp.float32) - jnp.float32(1.0)
    return jnp.log(-jnp.log(fl + TINY))


G_BOUND = np.float32(16.0)


def _argmax_kernel(t_ref, approx_ref, perm_ref, cmax_ref, logp_ref, out_ref):
    n = pl.program_id(0)
    row_base = n * (L_TOK * (NUM_BINS - 1))

    @pl.when(approx_ref[n] == 0)
    def _():
        l_iota = jax.lax.broadcasted_iota(jnp.int32, (L_TOK, CHUNK), 0)
        j_iota = jax.lax.broadcasted_iota(jnp.int32, (L_TOK, CHUNK), 1)
        tok_base = (row_base + KS1_I32) + (
            jax.lax.shift_left(l_iota, 13) - l_iota)

        def cond(state):
            i, _, _, rowmin = state
            ii = jnp.minimum(i, N_CHUNKS - 1)
            return jnp.logical_and(i < N_CHUNKS,
                                   cmax_ref[n * N_CHUNKS + ii] + G_BOUND > rowmin)

        def body(state):
            i, best_v, best_i, _ = state
            j0 = perm_ref[n * N_CHUNKS + i] * CHUNK
            jglob = j0 + j_iota
            l2 = _neg_gumbel_from_x1((tok_base + jglob).astype(jnp.uint32))
            logp = logp_ref[0, 0, pl.ds(j0, CHUNK)]
            v = logp[None, :] - l2
            cmax = jnp.max(v, axis=1, keepdims=True)
            cidx = jnp.min(jnp.where(v == cmax, jglob, np.int32(1 << 30)),
                           axis=1, keepdims=True)
            upd = jnp.logical_or(
                cmax > best_v,
                jnp.logical_and(cmax == best_v, cidx < best_i))
            best_v = jnp.where(upd, cmax, best_v)
            best_i = jnp.where(upd, cidx, best_i)
            return (i + 1, best_v, best_i, jnp.min(best_v))

        best_v0 = jnp.full((L_TOK, 1), NEG_INF, jnp.float32)
        best_i0 = jnp.full((L_TOK, 1), np.int32(1 << 30), jnp.int32)
        state = (jnp.int32(0), best_v0, best_i0, NEG_INF)
        _, _, best_i, _ = jax.lax.while_loop(cond, body, state)
        out_ref[0] = best_i


def _bf16(x):
    return x.astype(jnp.bfloat16).astype(jnp.float32)


def _rot_from_vec(wx, wy, wz):
    th = jnp.sqrt(wx * wx + wy * wy + wz * wz)
    x = wx / (th + 1e-12)
    y = wy / (th + 1e-12)
    z = wz / (th + 1e-12)
    bx = _bf16(x)
    by = _bf16(y)
    bz = _bf16(z)
    s = jnp.sin(th)
    c1 = 1.0 - jnp.cos(th)
    b00 = (-(bz * bz)) + (-(by * by))
    b11 = (-(bz * bz)) + (-(bx * bx))
    b22 = (-(by * by)) + (-(bx * bx))
    bxy = bx * by
    bxz = bx * bz
    byz = by * bz
    r00 = 1.0 + c1 * b00
    r01 = (-(s * z)) + c1 * bxy
    r02 = (s * y) + c1 * bxz
    r10 = (s * z) + c1 * bxy
    r11 = 1.0 + c1 * b11
    r12 = (-(s * x)) + c1 * byz
    r20 = (-(s * y)) + c1 * bxz
    r21 = (s * x) + c1 * byz
    r22 = 1.0 + c1 * b22
    return (r00, r01, r02, r10, r11, r12, r20, r21, r22)


def _tail_kernel(bin_ref, ux_ref, uy_ref, uz_ref, unif_ref, gau_ref,
                 v0x_ref, v0y_ref, v0z_ref, std_ref, c0_ref, approx_ref,
                 mask_ref,
                 vnx_ref, vny_ref, vnz_ref, ex_ref, ey_ref, ez_ref):
    bin_idx = bin_ref[:, 0, :]
    idx_f = bin_idx.astype(jnp.float32)
    bs = idx_f * STEP
    bw = (idx_f + 1.0) * STEP - bs
    theta_hist = bs + unif_ref[...] * bw

    std = std_ref[...]
    sg = jnp.abs(2.0 * std + gau_ref[...] * std)
    theta_gauss = sg - jnp.floor(sg * np.float32(1.0 / PI)) * np.float32(PI)
    theta = jnp.where(approx_ref[...] != 0, theta_gauss, theta_hist)

    ux = ux_ref[...]
    uy = uy_ref[...]
    uz = uz_ref[...]
    un = jnp.sqrt(ux * ux + uy * uy + uz * uz) + 1e-12
    scale = theta / un
    ex = ux * scale
    ey = uy * scale
    ez = uz * scale
    ex_ref[...] = ex
    ey_ref[...] = ey
    ez_ref[...] = ez

    e00, e01, e02, e10, e11, e12, e20, e21, e22 = _rot_from_vec(ex, ey, ez)

    c0 = c0_ref[...]
    w0x = c0 * v0x_ref[...]
    w0y = c0 * v0y_ref[...]
    w0z = c0 * v0z_ref[...]
    a00, a01, a02, a10, a11, a12, a20, a21, a22 = _rot_from_vec(w0x, w0y, w0z)

    a00, a01, a02 = _bf16(a00), _bf16(a01), _bf16(a02)
    a10, a11, a12 = _bf16(a10), _bf16(a11), _bf16(a12)
    a20, a21, a22 = _bf16(a20), _bf16(a21), _bf16(a22)
    e00b, e01b, e02b = _bf16(e00), _bf16(e01), _bf16(e02)
    e10b, e11b, e12b = _bf16(e10), _bf16(e11), _bf16(e12)
    e20b, e21b, e22b = _bf16(e20), _bf16(e21), _bf16(e22)

    m00 = (a00 * e00b + a01 * e10b) + a02 * e20b
    m01 = (a00 * e01b + a01 * e11b) + a02 * e21b
    m02 = (a00 * e02b + a01 * e12b) + a02 * e22b
    m10 = (a10 * e00b + a11 * e10b) + a12 * e20b
    m11 = (a10 * e01b + a11 * e11b) + a12 * e21b
    m12 = (a10 * e02b + a11 * e12b) + a12 * e22b
    m20 = (a20 * e00b + a21 * e10b) + a22 * e20b
    m21 = (a20 * e01b + a21 * e11b) + a22 * e21b
    m22 = (a20 * e02b + a21 * e12b) + a22 * e22b

    tr = m00 + m11 + m22
    cos_t = jnp.clip((tr - 1.0) * 0.5, np.float32(-1.0 + 1e-7),
                     np.float32(1.0 - 1e-7))
    th_n = 2.0 * jnp.arctan2(jnp.sqrt(1.0 - cos_t * cos_t), 1.0 + cos_t)
    fac = th_n / (2.0 * jnp.sin(th_n) + 1e-12)
    vnx = (m21 - m12) * fac
    vny = (m02 - m20) * fac
    vnz = (m10 - m01) * fac

    keep = mask_ref[...] != 0
    vnx_ref[...] = jnp.where(keep, vnx, v0x_ref[...])
    vny_ref[...] = jnp.where(keep, vny, v0y_ref[...])
    vnz_ref[...] = jnp.where(keep, vnz, v0z_ref[...])


def kernel(v0, generation_mask, t, alpha_bars, X, hist, stddevs, approx_mask):
    N, L = generation_mask.shape
    t = t.astype(jnp.int32)

    key = jax.random.key(42)
    k1, k2 = jax.random.split(key)
    _, k22, k23 = jax.random.split(k2, 3)
    u = jax.random.normal(k1, (N, L, 3), dtype=jnp.float32)
    unif = jax.random.uniform(k22, (N * L,), jnp.float32).reshape(N, L)
    gau = jax.random.normal(k23, (N * L,), jnp.float32).reshape(N, L)

    logp = jnp.log(hist + 1e-30)
    logp = logp.at[:, NUM_BINS - 1].set(NEG_INF)

    chunk_max = jnp.max(logp.reshape(logp.shape[0], N_CHUNKS, CHUNK), axis=2)
    order = jnp.argsort(-chunk_max, axis=1).astype(jnp.int32)
    cmax_sorted = jnp.take_along_axis(chunk_max, order, axis=1)
    perm_rows = order[t].reshape(N * N_CHUNKS)
    cmax_rows = cmax_sorted[t].reshape(N * N_CHUNKS)

    approx_row = approx_mask[t].astype(jnp.int32)

    grid_spec = pltpu.PrefetchScalarGridSpec(
        num_scalar_prefetch=4,
        grid=(N,),
        in_specs=[
            pl.BlockSpec((1, 1, NUM_BINS),
                         lambda n, t_ref, a_ref, p_ref, c_ref: (t_ref[n], 0, 0)),
        ],
        out_specs=pl.BlockSpec((1, L_TOK, 1),
                               lambda n, t_ref, a_ref, p_ref, c_ref: (n, 0, 0)),
    )
    bin_idx = pl.pallas_call(
        _argmax_kernel,
        grid_spec=grid_spec,
        out_shape=jax.ShapeDtypeStruct((N, L, 1), jnp.int32),
    )(t, approx_row, perm_rows, cmax_rows,
      logp.reshape(logp.shape[0], 1, NUM_BINS))
    bin_idx = bin_idx.reshape(N, 1, L)

    std_b = jnp.broadcast_to(stddevs[t][:, None], (N, L))
    c0_b = jnp.broadcast_to(jnp.sqrt(alpha_bars[t])[:, None], (N, L))
    approx_b = jnp.broadcast_to(approx_row[:, None], (N, L))
    mask_b = generation_mask.astype(jnp.int32)

    outs = pl.pallas_call(
        _tail_kernel,
        out_shape=[jax.ShapeDtypeStruct((N, L), jnp.float32)] * 6,
    )(bin_idx, u[..., 0], u[..., 1], u[..., 2], unif, gau,
      v0[..., 0], v0[..., 1], v0[..., 2], std_b, c0_b, approx_b, mask_b)

    vnx, vny, vnz, ex, ey, ez = outs
    v_noisy = jnp.stack([vnx, vny, vnz], axis=-1)
    e_scaled = jnp.stack([ex, ey, ez], axis=-1)
    return (v_noisy, e_scaled)
